# Initial kernel scaffold; baseline (speedup 1.0000x reference)
import jax
import jax.numpy as jnp
from jax.experimental import pallas as pl


def kernel(x, edge_index, lin_w0, lin_b0, w_w0, w_b0, beta0, lin_w1, lin_b1, w_w1, w_b1, beta1):
    raise NotImplementedError("write your pallas kernel here")


# speedup vs baseline: 6.5960x; 6.5960x over previous
import functools
import jax
import jax.numpy as jnp
import numpy as np
from jax import lax
from jax.experimental import pallas as pl
from jax.experimental.pallas import tpu as pltpu
from jax.experimental.pallas import tpu_sc as plsc

N = 10000
E = 320000
NW = 32
EPW = E // NW
CH = 80
NCH = EPW // CH
INT_MAX = np.int32(2**31 - 1)
NEG_INF = np.float32(-np.inf)
BIG_E = np.float32(2.0**30)
ACC_ROWS = 10112
DUMMY_ROW = N
ND_L = 336
ND_SP = 312


def _mesh():
    return plsc.VectorSubcoreMesh(core_axis_name="c", subcore_axis_name="s")


def _wid():
    return lax.axis_index("s") * 2 + lax.axis_index("c")


def _tc_linnorm(x, w, b):
    n, in_c = x.shape
    h = w.shape[0]
    blk = 1000

    def kern(x_ref, w_ref, b_ref, xl_ref, nrm_ref):
        xl = lax.dot_general(x_ref[...], w_ref[...], (((1,), (1,)), ((), ())),
                             preferred_element_type=jnp.float32) + b_ref[...]
        nn = jnp.sqrt(jnp.sum(xl * xl, axis=1, keepdims=True))
        nrm = xl / jnp.maximum(nn, 1e-12)
        if h < 128:
            z = jnp.zeros((blk, 128 - h), jnp.float32)
            xl = jnp.concatenate([xl, z], axis=1)
            nrm = jnp.concatenate([nrm, z], axis=1)
        xl_ref[...] = xl
        nrm_ref[...] = nrm

    return pl.pallas_call(
        kern,
        grid=(n // blk,),
        in_specs=[
            pl.BlockSpec((blk, in_c), lambda i: (i, 0)),
            pl.BlockSpec((h, in_c), lambda i: (0, 0)),
            pl.BlockSpec((1, h), lambda i: (0, 0)),
        ],
        out_specs=[
            pl.BlockSpec((blk, 128), lambda i: (i, 0)),
            pl.BlockSpec((blk, 128), lambda i: (i, 0)),
        ],
        out_shape=[
            jax.ShapeDtypeStruct((n, 128), jnp.float32),
            jax.ShapeDtypeStruct((n, 128), jnp.float32),
        ],
    )(x, w.reshape(h, in_c), b.reshape(1, h))


def _lanesum(v, redbuf):
    s1 = v + lax.rev(v, (0,))
    redbuf[pl.ds(0, 16)] = s1
    a = s1 + redbuf[pl.ds(4, 16)]
    redbuf[pl.ds(16, 16)] = a
    b = a + redbuf[pl.ds(18, 16)]
    redbuf[pl.ds(32, 16)] = b
    return (b + redbuf[pl.ds(33, 16)])[0]


TR = 5008


def _sc_sim_select(nrm, src, dst, h):
    nsl = h // 16

    def kern(nrm_hbm, src_hbm, dst_hbm, tab_o, smin_o,
             tab, sidx, didx, srows, drows, sminb, redbuf, sem1, sem2):
        wid = _wid()
        lanes = lax.broadcasted_iota(jnp.int32, (16,), 0)
        l7 = lanes & 7
        initrow = jnp.where((l7 == 0) | (l7 == 3), NEG_INF,
                            jnp.where((l7 == 1) | (l7 == 4), BIG_E,
                                      0.0)).astype(jnp.float32)

        def init_body(i, carry):
            tab[pl.ds(i * 16, 16)] = initrow
            return carry

        lax.fori_loop(0, TR, init_body, 0)

        def chunk_body(c, smin16):
            base = wid * EPW + c * CH
            pltpu.sync_copy(src_hbm.at[pl.ds(base, CH)], sidx.at[pl.ds(0, CH)])
            pltpu.sync_copy(dst_hbm.at[pl.ds(base, CH)], didx.at[pl.ds(0, CH)])
            cp1 = pltpu.async_copy(nrm_hbm.at[sidx.at[pl.ds(0, CH)]],
                                   srows, sem1)
            cp2 = pltpu.async_copy(nrm_hbm.at[didx.at[pl.ds(0, CH)]],
                                   drows, sem2)
            cp1.wait()
            cp2.wait()

            def group_body(g, smin16):
                s16 = sidx[pl.ds(g * 16, 16)]
                d16 = didx[pl.ds(g * 16, 16)]
                keep16 = s16 != d16
                smin16 = jnp.minimum(smin16, jnp.where(keep16, s16, N))
                for j in range(16):
                    r = g * 16 + j
                    acc = srows[r, pl.ds(0, 16)] * drows[r, pl.ds(0, 16)]
                    for s in range(1, nsl):
                        acc = acc + (srows[r, pl.ds(s * 16, 16)] *
                                     drows[r, pl.ds(s * 16, 16)])
                    sim_j = _lanesum(acc, redbuf)
                    dj = didx[pl.ds(r, 16)][0]
                    sj = sidx[pl.ds(r, 16)][0]
                    kj = sj != dj
                    vj = jnp.where(kj, sim_j, NEG_INF)
                    ej = jnp.where(kj, (base + r).astype(jnp.float32), BIG_E)
                    sjf = sj.astype(jnp.float32)
                    p = dj * 8
                    w0 = tab[pl.ds(p, 16)]
                    c1v = w0[0]
                    c1e = tab[pl.ds(p + 1, 16)][0]
                    c1s = tab[pl.ds(p + 2, 16)][0]
                    c2v = tab[pl.ds(p + 3, 16)][0]
                    c2e = tab[pl.ds(p + 4, 16)][0]
                    c2s = tab[pl.ds(p + 5, 16)][0]
                    gt1 = (vj > c1v) | ((vj == c1v) & (ej < c1e))
                    gt2 = (vj > c2v) | ((vj == c2v) & (ej < c2e))
                    n1v = jnp.where(gt1, vj, c1v)
                    n1e = jnp.where(gt1, ej, c1e)
                    n1s = jnp.where(gt1, sjf, c1s)
                    n2v = jnp.where(gt1, c1v, jnp.where(gt2, vj, c2v))
                    n2e = jnp.where(gt1, c1e, jnp.where(gt2, ej, c2e))
                    n2s = jnp.where(gt1, c1s, jnp.where(gt2, sjf, c2s))
                    kadd = jnp.where(kj, 1.0, 0.0).astype(jnp.float32)
                    nrow = jnp.where(lanes == 0, n1v, w0)
                    nrow = jnp.where(lanes == 1, n1e, nrow)
                    nrow = jnp.where(lanes == 2, n1s, nrow)
                    nrow = jnp.where(lanes == 3, n2v, nrow)
                    nrow = jnp.where(lanes == 4, n2e, nrow)
                    nrow = jnp.where(lanes == 5, n2s, nrow)
                    nrow = jnp.where(lanes == 6, w0 + kadd, nrow)
                    tab[pl.ds(p, 16)] = nrow
                return smin16

            return lax.fori_loop(0, CH // 16, group_body, smin16)

        smin16 = lax.fori_loop(0, NCH, chunk_body,
                               jnp.full((16,), N, jnp.int32))
        sminb[...] = smin16
        pltpu.sync_copy(tab.at[pl.ds(0, TR * 16)], tab_o.at[wid])
        pltpu.sync_copy(sminb, smin_o.at[wid])

    f32 = jnp.float32
    i32 = jnp.int32
    out_type = [
        jax.ShapeDtypeStruct((NW, TR * 16), f32),
        jax.ShapeDtypeStruct((NW, 16), i32),
    ]
    scratch = [
        pltpu.VMEM((TR * 16 + 16,), f32),
        pltpu.VMEM((CH + 16,), i32), pltpu.VMEM((CH + 16,), i32),
        pltpu.VMEM((CH, 128), f32), pltpu.VMEM((CH, 128), f32),
        pltpu.VMEM((16,), i32),
        pltpu.VMEM((64,), f32),
        pltpu.SemaphoreType.DMA, pltpu.SemaphoreType.DMA,
    ]
    fn = functools.partial(pl.kernel, mesh=_mesh(), out_type=out_type,
                           scratch_types=scratch)(kern)
    return fn(nrm, src, dst)


def _unpack_tab(tab):
    t = tab.reshape(NW, TR * 2, 8)[:, :N, :]
    return (t[:, :, 0], t[:, :, 1], t[:, :, 2],
            t[:, :, 3], t[:, :, 4], t[:, :, 5], t[:, :, 6])


def _tc_merge(b1v, b1e, b1s, b2v, b2e, b2s, cnt):
    blk = N

    def kern(b1v_r, b1e_r, b1s_r, b2v_r, b2e_r, b2s_r, cnt_r,
             w1_o, s1_o, w2_o, s2_o, cnt_o):
        a1v = b1v_r[pl.ds(0, 1), :]
        a1e = b1e_r[pl.ds(0, 1), :]
        a1s = b1s_r[pl.ds(0, 1), :]
        a2v = b2v_r[pl.ds(0, 1), :]
        a2e = b2e_r[pl.ds(0, 1), :]
        a2s = b2s_r[pl.ds(0, 1), :]

        def body(j, carry):
            a1v, a1e, a1s, a2v, a2e, a2s = carry
            c1v = b1v_r[pl.ds(j, 1), :]
            c1e = b1e_r[pl.ds(j, 1), :]
            c1s = b1s_r[pl.ds(j, 1), :]
            c2v = b2v_r[pl.ds(j, 1), :]
            c2e = b2e_r[pl.ds(j, 1), :]
            c2s = b2s_r[pl.ds(j, 1), :]
            gA = (a1v > c1v) | ((a1v == c1v) & (a1e < c1e))
            n1v = jnp.where(gA, a1v, c1v)
            n1e = jnp.where(gA, a1e, c1e)
            n1s = jnp.where(gA, a1s, c1s)
            l1v = jnp.where(gA, c1v, a1v)
            l1e = jnp.where(gA, c1e, a1e)
            l1s = jnp.where(gA, c1s, a1s)
            p2v = jnp.where(gA, a2v, c2v)
            p2e = jnp.where(gA, a2e, c2e)
            p2s = jnp.where(gA, a2s, c2s)
            gB = (l1v > p2v) | ((l1v == p2v) & (l1e < p2e))
            n2v = jnp.where(gB, l1v, p2v)
            n2e = jnp.where(gB, l1e, p2e)
            n2s = jnp.where(gB, l1s, p2s)
            return (n1v, n1e, n1s, n2v, n2e, n2s)

        a1v, a1e, a1s, a2v, a2e, a2s = lax.fori_loop(
            1, NW, body, (a1v, a1e, a1s, a2v, a2e, a2s))
        val1 = (a1e < BIG_E) & (a1v >= 0.0)
        val2 = (a2e < BIG_E) & (a2v >= 0.0)
        w1_o[...] = jnp.where(val1, a1v, 0.0)
        s1_o[...] = jnp.where(val1, a1s, 0.0).astype(jnp.int32)
        w2_o[...] = jnp.where(val2, a2v, 0.0)
        s2_o[...] = jnp.where(val2, a2s, 0.0).astype(jnp.int32)
        cnt_o[...] = jnp.sum(cnt_r[...], axis=0, keepdims=True)

    f32 = jnp.float32
    i32 = jnp.int32
    io = pl.BlockSpec((NW, blk), lambda i: (0, 0))
    oo = pl.BlockSpec((1, blk), lambda i: (0, 0))
    return pl.pallas_call(
        kern,
        grid=(N // blk,),
        in_specs=[io] * 7,
        out_specs=[oo] * 5,
        out_shape=[
            jax.ShapeDtypeStruct((1, N), f32),
            jax.ShapeDtypeStruct((1, N), i32),
            jax.ShapeDtypeStruct((1, N), f32),
            jax.ShapeDtypeStruct((1, N), i32),
            jax.ShapeDtypeStruct((1, N), f32),
        ],
    )(b1v, b1e, b1s, b2v, b2e, b2s, cnt)


def _tc_rowidx(src2d, dst2d, smin):
    blk = src2d.shape[0]

    def kern(src_r, dst_r, smin_r, row_o):
        sm = jnp.min(smin_r[...])
        s = src_r[...]
        d = dst_r[...]
        row_o[...] = jnp.where(s != d, s - sm, DUMMY_ROW)

    return pl.pallas_call(
        kern,
        grid=(src2d.shape[0] // blk,),
        in_specs=[
            pl.BlockSpec((blk, 128), lambda i: (i, 0)),
            pl.BlockSpec((blk, 128), lambda i: (i, 0)),
            pl.BlockSpec((NW, 16), lambda i: (0, 0)),
        ],
        out_specs=pl.BlockSpec((blk, 128), lambda i: (i, 0)),
        out_shape=jax.ShapeDtypeStruct(src2d.shape, jnp.int32),
    )(src2d, dst2d, smin)


def _sc_out0(w0t, w1tp, dst, row):
    hp = 128
    zrows = 104
    rows_per_tile = ACC_ROWS // 16
    epw = E // 16
    nch = epw // CH

    def kern(w0_hbm, w1_hbm, dst_hbm, row_hbm, out_hbm,
             didx, ridx, wrows, zbuf, acc, sem1):
        cid = lax.axis_index("c")
        sid = lax.axis_index("s")

        def zinit(i, carry):
            r = i // (hp // 16)
            cc = (i % (hp // 16)) * 16
            zbuf[r, pl.ds(cc, 16)] = jnp.zeros((16,), jnp.float32)
            return carry

        lax.fori_loop(0, zrows * (hp // 16), zinit, 0)
        zb = sid * rows_per_tile
        for t in range(6):
            pltpu.sync_copy(zbuf, acc.at[pl.ds(zb + t * zrows, zrows)])
        pltpu.sync_copy(zbuf.at[pl.ds(0, 8)],
                        acc.at[pl.ds(zb + 6 * zrows, 8)])
        plsc.subcore_barrier()

        def make_body(w_hbm):
            def chunk_body(c, carry):
                base = sid * epw + c * CH
                pltpu.sync_copy(dst_hbm.at[pl.ds(base, CH)], didx)
                pltpu.sync_copy(row_hbm.at[pl.ds(base, CH)], ridx)
                pltpu.async_copy(w_hbm.at[didx], wrows, sem1).wait()
                pltpu.sync_copy(wrows, acc.at[ridx], add=True)
                return carry
            return chunk_body

        @pl.when(cid == 0)
        def _():
            lax.fori_loop(0, nch, make_body(w0_hbm), 0)

        @pl.when(cid == 1)
        def _():
            lax.fori_loop(0, nch, make_body(w1_hbm), 0)

        plsc.subcore_barrier()

        @pl.when(sid == 0)
        def _():
            pltpu.sync_copy(acc, out_hbm.at[cid])

    out_type = jax.ShapeDtypeStruct((2, ACC_ROWS, hp), jnp.float32)
    scratch = [
        pltpu.VMEM((CH,), jnp.int32), pltpu.VMEM((CH,), jnp.int32),
        pltpu.VMEM((CH, hp), jnp.float32),
        pltpu.VMEM((zrows, hp), jnp.float32),
        pltpu.VMEM_SHARED((ACC_ROWS, hp), jnp.float32),
        pltpu.SemaphoreType.DMA,
    ]
    fn = functools.partial(pl.kernel, mesh=_mesh(), out_type=out_type,
                           scratch_types=scratch)(kern)
    return fn(w0t, w1tp, dst, row)


def _sc_gather_rows(xl, s1, s2):
    def kern(xl_hbm, s1_hbm, s2_hbm, g1_o, g2_o,
             i1, i2, r1, r2, sem1, sem2):
        wid = _wid()
        base = jnp.where(wid == NW - 1, N - ND_L, wid * ND_SP)
        pltpu.sync_copy(s1_hbm.at[pl.ds(base, ND_L)], i1)
        pltpu.sync_copy(s2_hbm.at[pl.ds(base, ND_L)], i2)
        cp1 = pltpu.async_copy(xl_hbm.at[i1], r1, sem1)
        cp2 = pltpu.async_copy(xl_hbm.at[i2], r2, sem2)
        cp1.wait()
        cp2.wait()
        pltpu.sync_copy(r1, g1_o.at[pl.ds(base, ND_L)])
        pltpu.sync_copy(r2, g2_o.at[pl.ds(base, ND_L)])

    out_type = [
        jax.ShapeDtypeStruct((N, 128), jnp.float32),
        jax.ShapeDtypeStruct((N, 128), jnp.float32),
    ]
    scratch = [
        pltpu.VMEM((ND_L,), jnp.int32), pltpu.VMEM((ND_L,), jnp.int32),
        pltpu.VMEM((ND_L, 128), jnp.float32),
        pltpu.VMEM((ND_L, 128), jnp.float32),
        pltpu.SemaphoreType.DMA, pltpu.SemaphoreType.DMA,
    ]
    fn = functools.partial(pl.kernel, mesh=_mesh(), out_type=out_type,
                           scratch_types=scratch)(kern)
    return fn(xl, s1, s2)


def _tc_combine(p, g1, g2, w1c, w2c, cntc, w_b, beta, plane, act):
    h = w_b.shape[1]
    hcat = p.shape[2]
    blk = 2000

    def kern(p_r, g1_r, g2_r, w1_r, w2_r, cnt_r, wb_r, beta_r, o_ref):
        out0 = p_r[plane, :, 0:h] + wb_r[...]
        num = (w1_r[...] * g1_r[:, 0:h] + w2_r[...] * g2_r[:, 0:h])
        out1 = num / jnp.maximum(cnt_r[...], 1.0)
        b = beta_r[0, 0]
        hh = b * out0 + (1.0 - b) * out1
        if act == "relu":
            o_ref[...] = jnp.maximum(hh, 0.0)
        else:
            m = jnp.max(hh, axis=1, keepdims=True)
            ex = jnp.exp(hh - m)
            o_ref[...] = (hh - m) - jnp.log(jnp.sum(ex, axis=1, keepdims=True))

    return pl.pallas_call(
        kern,
        grid=(N // blk,),
        in_specs=[
            pl.BlockSpec((2, blk, hcat), lambda i: (0, i, 0)),
            pl.BlockSpec((blk, 128), lambda i: (i, 0)),
            pl.BlockSpec((blk, 128), lambda i: (i, 0)),
            pl.BlockSpec((blk, 1), lambda i: (i, 0)),
            pl.BlockSpec((blk, 1), lambda i: (i, 0)),
            pl.BlockSpec((blk, 1), lambda i: (i, 0)),
            pl.BlockSpec((1, h), lambda i: (0, 0)),
            pl.BlockSpec((1, 1), lambda i: (0, 0)),
        ],
        out_specs=pl.BlockSpec((blk, h), lambda i: (i, 0)),
        out_shape=jax.ShapeDtypeStruct((N, h), jnp.float32),
    )(p, g1, g2, w1c, w2c, cntc, w_b, beta)


def kernel(x, edge_index, lin_w0, lin_b0, w_w0, w_b0, beta0,
           lin_w1, lin_b1, w_w1, w_b1, beta1):
    src = edge_index[0]
    dst = edge_index[1]
    h0 = lin_w0.shape[0]
    h1 = lin_w1.shape[0]

    xl0, nrm0 = _tc_linnorm(x, lin_w0, lin_b0)
    tab0, smin = _sc_sim_select(nrm0, src, dst, h0)
    b1v, b1e, b1s, b2v, b2e, b2s, cnt = _unpack_tab(tab0)
    w1, s1, w2, s2, cnt_m = _tc_merge(b1v, b1e, b1s, b2v, b2e, b2s, cnt)
    cntc = cnt_m.reshape(N, 1)

    row2d = _tc_rowidx(src.reshape(E // 128, 128), dst.reshape(E // 128, 128),
                       smin)
    w1tp = jnp.pad(w_w1.T, ((0, 0), (0, 128 - h1)))
    p = _sc_out0(w_w0.T, w1tp, dst, row2d.reshape(E))

    g1, g2 = _sc_gather_rows(xl0, s1.reshape(N), s2.reshape(N))
    hmid = _tc_combine(p, g1, g2, w1.reshape(N, 1), w2.reshape(N, 1), cntc,
                       w_b0.reshape(1, h0), beta0.reshape(1, 1), 0, "relu")

    xl1, nrm1 = _tc_linnorm(hmid, lin_w1, lin_b1)
    tab1, _ = _sc_sim_select(nrm1, src, dst, h1)
    c1v, c1e, c1s, c2v, c2e, c2s, _ = _unpack_tab(tab1)
    v1, t1, v2, t2, _ = _tc_merge(c1v, c1e, c1s, c2v, c2e, c2s, cnt)
    k1, k2 = _sc_gather_rows(xl1, t1.reshape(N), t2.reshape(N))
    out = _tc_combine(p, k1, k2, v1.reshape(N, 1), v2.reshape(N, 1), cntc,
                      w_b1.reshape(1, h1), beta1.reshape(1, 1),
                      1, "softmax")
    return out


# speedup vs baseline: 8.9838x; 1.3620x over previous
import functools
import jax
import jax.numpy as jnp
import numpy as np
from jax import lax
from jax.experimental import pallas as pl
from jax.experimental.pallas import tpu as pltpu
from jax.experimental.pallas import tpu_sc as plsc

N = 10000
E = 320000
NW = 32
EPW = E // NW
CH = 80
NCH = EPW // CH
SBE = 2000
SBC = SBE // CH
INT_MAX = np.int32(2**31 - 1)
NEG_INF = np.float32(-np.inf)
BIG_E = np.float32(2.0**30)
ACC_ROWS = 10112
DUMMY_ROW = N
ND_L = 336
ND_SP = 312


def _mesh():
    return plsc.VectorSubcoreMesh(core_axis_name="c", subcore_axis_name="s")


def _wid():
    return lax.axis_index("s") * 2 + lax.axis_index("c")


def _tc_linnorm(x, w, b):
    n, in_c = x.shape
    h = w.shape[0]
    blk = 1000

    def kern(x_ref, w_ref, b_ref, xl_ref, nrm_ref):
        xl = lax.dot_general(x_ref[...], w_ref[...], (((1,), (1,)), ((), ())),
                             preferred_element_type=jnp.float32) + b_ref[...]
        nn = jnp.sqrt(jnp.sum(xl * xl, axis=1, keepdims=True))
        nrm = xl / jnp.maximum(nn, 1e-12)
        if h < 128:
            z = jnp.zeros((blk, 128 - h), jnp.float32)
            xl = jnp.concatenate([xl, z], axis=1)
            nrm = jnp.concatenate([nrm, z], axis=1)
        xl_ref[...] = xl
        nrm_ref[...] = nrm

    return pl.pallas_call(
        kern,
        grid=(n // blk,),
        in_specs=[
            pl.BlockSpec((blk, in_c), lambda i: (i, 0)),
            pl.BlockSpec((h, in_c), lambda i: (0, 0)),
            pl.BlockSpec((1, h), lambda i: (0, 0)),
        ],
        out_specs=[
            pl.BlockSpec((blk, 128), lambda i: (i, 0)),
            pl.BlockSpec((blk, 128), lambda i: (i, 0)),
        ],
        out_shape=[
            jax.ShapeDtypeStruct((n, 128), jnp.float32),
            jax.ShapeDtypeStruct((n, 128), jnp.float32),
        ],
    )(x, w.reshape(h, in_c), b.reshape(1, h))


def _lanesum(v, redbuf):
    s1 = v + lax.rev(v, (0,))
    redbuf[pl.ds(0, 16)] = s1
    a = s1 + redbuf[pl.ds(4, 16)]
    redbuf[pl.ds(16, 16)] = a
    b = a + redbuf[pl.ds(18, 16)]
    redbuf[pl.ds(32, 16)] = b
    return (b + redbuf[pl.ds(33, 16)])[0]


TR = 5008


def _sc_sim_select(nrm, src, dst, h):
    nsl = h // 16

    def kern(nrm_hbm, src_hbm, dst_hbm, tab_o, smin_o,
             tab, sidx, didx, srowsA, drowsA, srowsB, drowsB,
             sminb, redbuf, semAs, semAd, semBs, semBd):
        wid = _wid()
        lanes = lax.broadcasted_iota(jnp.int32, (16,), 0)
        l7 = lanes & 7
        initrow = jnp.where((l7 == 0) | (l7 == 3), NEG_INF,
                            jnp.where((l7 == 1) | (l7 == 4), BIG_E,
                                      0.0)).astype(jnp.float32)

        def init_body(i, carry):
            tab[pl.ds(i * 16, 16)] = initrow
            return carry

        lax.fori_loop(0, TR, init_body, 0)

        def issue(c, srows, drows, sems, semd):
            sl = pl.ds(c * CH, CH)
            pltpu.async_copy(nrm_hbm.at[sidx.at[sl]], srows, sems)
            pltpu.async_copy(nrm_hbm.at[didx.at[sl]], drows, semd)

        def wait(c, srows, drows, sems, semd):
            sl = pl.ds(c * CH, CH)
            pltpu.make_async_copy(nrm_hbm.at[sidx.at[sl]], srows, sems).wait()
            pltpu.make_async_copy(nrm_hbm.at[didx.at[sl]], drows, semd).wait()

        def compute(c, ebase, srows, drows, smin16):
            def group_body(g, smin16):
                s16 = sidx[pl.ds(c * CH + g * 16, 16)]
                d16 = didx[pl.ds(c * CH + g * 16, 16)]
                keep16 = s16 != d16
                smin16 = jnp.minimum(smin16, jnp.where(keep16, s16, N))
                for j in range(16):
                    r = g * 16 + j
                    acc = srows[r, pl.ds(0, 16)] * drows[r, pl.ds(0, 16)]
                    for s in range(1, nsl):
                        acc = acc + (srows[r, pl.ds(s * 16, 16)] *
                                     drows[r, pl.ds(s * 16, 16)])
                    sim_j = _lanesum(acc, redbuf)
                    dj = didx[pl.ds(c * CH + r, 16)][0]
                    sj = sidx[pl.ds(c * CH + r, 16)][0]
                    kj = sj != dj
                    vj = jnp.where(kj, sim_j, NEG_INF)
                    ej = jnp.where(kj, (ebase + c * CH + r)
                                   .astype(jnp.float32), BIG_E)
                    sjf = sj.astype(jnp.float32)
                    p = dj * 8
                    w0 = tab[pl.ds(p, 16)]
                    c1v = w0[0]
                    c1e = tab[pl.ds(p + 1, 16)][0]
                    c1s = tab[pl.ds(p + 2, 16)][0]
                    c2v = tab[pl.ds(p + 3, 16)][0]
                    c2e = tab[pl.ds(p + 4, 16)][0]
                    c2s = tab[pl.ds(p + 5, 16)][0]
                    gt1 = (vj > c1v) | ((vj == c1v) & (ej < c1e))
                    gt2 = (vj > c2v) | ((vj == c2v) & (ej < c2e))
                    n1v = jnp.where(gt1, vj, c1v)
                    n1e = jnp.where(gt1, ej, c1e)
                    n1s = jnp.where(gt1, sjf, c1s)
                    n2v = jnp.where(gt1, c1v, jnp.where(gt2, vj, c2v))
                    n2e = jnp.where(gt1, c1e, jnp.where(gt2, ej, c2e))
                    n2s = jnp.where(gt1, c1s, jnp.where(gt2, sjf, c2s))
                    kadd = jnp.where(kj, 1.0, 0.0).astype(jnp.float32)
                    nrow = jnp.where(lanes == 0, n1v, w0)
                    nrow = jnp.where(lanes == 1, n1e, nrow)
                    nrow = jnp.where(lanes == 2, n1s, nrow)
                    nrow = jnp.where(lanes == 3, n2v, nrow)
                    nrow = jnp.where(lanes == 4, n2e, nrow)
                    nrow = jnp.where(lanes == 5, n2s, nrow)
                    nrow = jnp.where(lanes == 6, w0 + kadd, nrow)
                    tab[pl.ds(p, 16)] = nrow
                return smin16

            return lax.fori_loop(0, CH // 16, group_body, smin16)

        def sb_body(sb, smin16):
            ebase = wid * EPW + sb * SBE
            pltpu.sync_copy(src_hbm.at[pl.ds(ebase, SBE)],
                            sidx.at[pl.ds(0, SBE)])
            pltpu.sync_copy(dst_hbm.at[pl.ds(ebase, SBE)],
                            didx.at[pl.ds(0, SBE)])
            issue(0, srowsA, drowsA, semAs, semAd)

            def pair_body(i, smin16):
                wait(2 * i, srowsA, drowsA, semAs, semAd)
                issue(2 * i + 1, srowsB, drowsB, semBs, semBd)
                smin16 = compute(2 * i, ebase, srowsA, drowsA, smin16)
                wait(2 * i + 1, srowsB, drowsB, semBs, semBd)
                issue(2 * i + 2, srowsA, drowsA, semAs, semAd)
                return compute(2 * i + 1, ebase, srowsB, drowsB, smin16)

            smin16 = lax.fori_loop(0, SBC // 2, pair_body, smin16)
            wait(SBC - 1, srowsA, drowsA, semAs, semAd)
            return compute(SBC - 1, ebase, srowsA, drowsA, smin16)

        smin16 = lax.fori_loop(0, EPW // SBE, sb_body,
                               jnp.full((16,), N, jnp.int32))
        sminb[...] = smin16
        pltpu.sync_copy(tab.at[pl.ds(0, TR * 16)], tab_o.at[wid])
        pltpu.sync_copy(sminb, smin_o.at[wid])

    f32 = jnp.float32
    i32 = jnp.int32
    out_type = [
        jax.ShapeDtypeStruct((NW, TR * 16), f32),
        jax.ShapeDtypeStruct((NW, 16), i32),
    ]
    scratch = [
        pltpu.VMEM((TR * 16 + 16,), f32),
        pltpu.VMEM((SBE + 16,), i32), pltpu.VMEM((SBE + 16,), i32),
        pltpu.VMEM((CH, 128), f32), pltpu.VMEM((CH, 128), f32),
        pltpu.VMEM((CH, 128), f32), pltpu.VMEM((CH, 128), f32),
        pltpu.VMEM((16,), i32),
        pltpu.VMEM((64,), f32),
        pltpu.SemaphoreType.DMA, pltpu.SemaphoreType.DMA,
        pltpu.SemaphoreType.DMA, pltpu.SemaphoreType.DMA,
    ]
    fn = functools.partial(pl.kernel, mesh=_mesh(), out_type=out_type,
                           scratch_types=scratch)(kern)
    return fn(nrm, src, dst)


def _unpack_tab(tab):
    t = tab.reshape(NW, TR * 2, 8)[:, :N, :]
    return (t[:, :, 0], t[:, :, 1], t[:, :, 2],
            t[:, :, 3], t[:, :, 4], t[:, :, 5], t[:, :, 6])


def _tc_merge(b1v, b1e, b1s, b2v, b2e, b2s, cnt):
    blk = N

    def kern(b1v_r, b1e_r, b1s_r, b2v_r, b2e_r, b2s_r, cnt_r,
             w1_o, s1_o, w2_o, s2_o, cnt_o):
        a1v = b1v_r[pl.ds(0, 1), :]
        a1e = b1e_r[pl.ds(0, 1), :]
        a1s = b1s_r[pl.ds(0, 1), :]
        a2v = b2v_r[pl.ds(0, 1), :]
        a2e = b2e_r[pl.ds(0, 1), :]
        a2s = b2s_r[pl.ds(0, 1), :]

        def body(j, carry):
            a1v, a1e, a1s, a2v, a2e, a2s = carry
            c1v = b1v_r[pl.ds(j, 1), :]
            c1e = b1e_r[pl.ds(j, 1), :]
            c1s = b1s_r[pl.ds(j, 1), :]
            c2v = b2v_r[pl.ds(j, 1), :]
            c2e = b2e_r[pl.ds(j, 1), :]
            c2s = b2s_r[pl.ds(j, 1), :]
            gA = (a1v > c1v) | ((a1v == c1v) & (a1e < c1e))
            n1v = jnp.where(gA, a1v, c1v)
            n1e = jnp.where(gA, a1e, c1e)
            n1s = jnp.where(gA, a1s, c1s)
            l1v = jnp.where(gA, c1v, a1v)
            l1e = jnp.where(gA, c1e, a1e)
            l1s = jnp.where(gA, c1s, a1s)
            p2v = jnp.where(gA, a2v, c2v)
            p2e = jnp.where(gA, a2e, c2e)
            p2s = jnp.where(gA, a2s, c2s)
            gB = (l1v > p2v) | ((l1v == p2v) & (l1e < p2e))
            n2v = jnp.where(gB, l1v, p2v)
            n2e = jnp.where(gB, l1e, p2e)
            n2s = jnp.where(gB, l1s, p2s)
            return (n1v, n1e, n1s, n2v, n2e, n2s)

        a1v, a1e, a1s, a2v, a2e, a2s = lax.fori_loop(
            1, NW, body, (a1v, a1e, a1s, a2v, a2e, a2s))
        val1 = (a1e < BIG_E) & (a1v >= 0.0)
        val2 = (a2e < BIG_E) & (a2v >= 0.0)
        w1_o[...] = jnp.where(val1, a1v, 0.0)
        s1_o[...] = jnp.where(val1, a1s, 0.0).astype(jnp.int32)
        w2_o[...] = jnp.where(val2, a2v, 0.0)
        s2_o[...] = jnp.where(val2, a2s, 0.0).astype(jnp.int32)
        cnt_o[...] = jnp.sum(cnt_r[...], axis=0, keepdims=True)

    f32 = jnp.float32
    i32 = jnp.int32
    io = pl.BlockSpec((NW, blk), lambda i: (0, 0))
    oo = pl.BlockSpec((1, blk), lambda i: (0, 0))
    return pl.pallas_call(
        kern,
        grid=(N // blk,),
        in_specs=[io] * 7,
        out_specs=[oo] * 5,
        out_shape=[
            jax.ShapeDtypeStruct((1, N), f32),
            jax.ShapeDtypeStruct((1, N), i32),
            jax.ShapeDtypeStruct((1, N), f32),
            jax.ShapeDtypeStruct((1, N), i32),
            jax.ShapeDtypeStruct((1, N), f32),
        ],
    )(b1v, b1e, b1s, b2v, b2e, b2s, cnt)


def _tc_rowidx(src2d, dst2d, smin):
    blk = src2d.shape[0]

    def kern(src_r, dst_r, smin_r, row_o):
        sm = jnp.min(smin_r[...])
        s = src_r[...]
        d = dst_r[...]
        row_o[...] = jnp.where(s != d, s - sm, DUMMY_ROW)

    return pl.pallas_call(
        kern,
        grid=(src2d.shape[0] // blk,),
        in_specs=[
            pl.BlockSpec((blk, 128), lambda i: (i, 0)),
            pl.BlockSpec((blk, 128), lambda i: (i, 0)),
            pl.BlockSpec((NW, 16), lambda i: (0, 0)),
        ],
        out_specs=pl.BlockSpec((blk, 128), lambda i: (i, 0)),
        out_shape=jax.ShapeDtypeStruct(src2d.shape, jnp.int32),
    )(src2d, dst2d, smin)


def _sc_out0(w0t, w1tp, dst, row):
    hp = 128
    zrows = 104
    rows_per_tile = ACC_ROWS // 16
    epw = E // 16
    CHO = 80
    SBO = 2000
    SBOC = SBO // CHO

    def kern(w0_hbm, w1_hbm, dst_hbm, row_hbm, out_hbm,
             didx, ridxA, ridxB, wrowsA, wrowsB, zbuf, acc,
             semAg, semAr, semBg, semBr):
        cid = lax.axis_index("c")
        sid = lax.axis_index("s")

        def zinit(i, carry):
            r = i // (hp // 16)
            cc = (i % (hp // 16)) * 16
            zbuf[r, pl.ds(cc, 16)] = jnp.zeros((16,), jnp.float32)
            return carry

        lax.fori_loop(0, zrows * (hp // 16), zinit, 0)
        zb = sid * rows_per_tile
        for t in range(6):
            pltpu.sync_copy(zbuf, acc.at[pl.ds(zb + t * zrows, zrows)])
        pltpu.sync_copy(zbuf.at[pl.ds(0, 8)],
                        acc.at[pl.ds(zb + 6 * zrows, 8)])
        plsc.subcore_barrier()

        def make_body(w_hbm):
            def issue(sbase, c, wrows, ridx, semg, semr):
                sl = pl.ds(c * CHO, CHO)
                pltpu.async_copy(w_hbm.at[didx.at[sl]], wrows, semg)
                pltpu.async_copy(row_hbm.at[pl.ds(sbase + c * CHO, CHO)],
                                 ridx, semr)

            def wait(sbase, c, wrows, ridx, semg, semr):
                sl = pl.ds(c * CHO, CHO)
                pltpu.make_async_copy(w_hbm.at[didx.at[sl]],
                                      wrows, semg).wait()
                pltpu.make_async_copy(row_hbm.at[pl.ds(sbase + c * CHO, CHO)],
                                      ridx, semr).wait()

            def scat(wrows, ridx):
                pltpu.sync_copy(wrows, acc.at[ridx], add=True)

            def sb_body(sb, carry):
                sbase = sid * epw + sb * SBO
                pltpu.sync_copy(dst_hbm.at[pl.ds(sbase, SBO)], didx)
                issue(sbase, 0, wrowsA, ridxA, semAg, semAr)

                def pair_body(i, carry):
                    wait(sbase, 2 * i, wrowsA, ridxA, semAg, semAr)
                    issue(sbase, 2 * i + 1, wrowsB, ridxB, semBg, semBr)
                    scat(wrowsA, ridxA)
                    wait(sbase, 2 * i + 1, wrowsB, ridxB, semBg, semBr)
                    issue(sbase, 2 * i + 2, wrowsA, ridxA, semAg, semAr)
                    scat(wrowsB, ridxB)
                    return carry

                lax.fori_loop(0, SBOC // 2, pair_body, 0)
                wait(sbase, SBOC - 1, wrowsA, ridxA, semAg, semAr)
                scat(wrowsA, ridxA)
                return carry

            lax.fori_loop(0, epw // SBO, sb_body, 0)

        @pl.when(cid == 0)
        def _():
            make_body(w0_hbm)

        @pl.when(cid == 1)
        def _():
            make_body(w1_hbm)

        plsc.subcore_barrier()

        @pl.when(sid == 0)
        def _():
            pltpu.sync_copy(acc, out_hbm.at[cid])

    out_type = jax.ShapeDtypeStruct((2, ACC_ROWS, hp), jnp.float32)
    scratch = [
        pltpu.VMEM((SBO,), jnp.int32),
        pltpu.VMEM((CHO,), jnp.int32), pltpu.VMEM((CHO,), jnp.int32),
        pltpu.VMEM((CHO, hp), jnp.float32), pltpu.VMEM((CHO, hp), jnp.float32),
        pltpu.VMEM((zrows, hp), jnp.float32),
        pltpu.VMEM_SHARED((ACC_ROWS, hp), jnp.float32),
        pltpu.SemaphoreType.DMA, pltpu.SemaphoreType.DMA,
        pltpu.SemaphoreType.DMA, pltpu.SemaphoreType.DMA,
    ]
    fn = functools.partial(pl.kernel, mesh=_mesh(), out_type=out_type,
                           scratch_types=scratch)(kern)
    return fn(w0t, w1tp, dst, row)


def _sc_gather_rows(xl, s1, s2):
    def kern(xl_hbm, s1_hbm, s2_hbm, g1_o, g2_o,
             i1, i2, r1, r2, sem1, sem2):
        wid = _wid()
        base = jnp.where(wid == NW - 1, N - ND_L, wid * ND_SP)
        pltpu.sync_copy(s1_hbm.at[pl.ds(base, ND_L)], i1)
        pltpu.sync_copy(s2_hbm.at[pl.ds(base, ND_L)], i2)
        cp1 = pltpu.async_copy(xl_hbm.at[i1], r1, sem1)
        cp2 = pltpu.async_copy(xl_hbm.at[i2], r2, sem2)
        cp1.wait()
        cp2.wait()
        pltpu.sync_copy(r1, g1_o.at[pl.ds(base, ND_L)])
        pltpu.sync_copy(r2, g2_o.at[pl.ds(base, ND_L)])

    out_type = [
        jax.ShapeDtypeStruct((N, 128), jnp.float32),
        jax.ShapeDtypeStruct((N, 128), jnp.float32),
    ]
    scratch = [
        pltpu.VMEM((ND_L,), jnp.int32), pltpu.VMEM((ND_L,), jnp.int32),
        pltpu.VMEM((ND_L, 128), jnp.float32),
        pltpu.VMEM((ND_L, 128), jnp.float32),
        pltpu.SemaphoreType.DMA, pltpu.SemaphoreType.DMA,
    ]
    fn = functools.partial(pl.kernel, mesh=_mesh(), out_type=out_type,
                           scratch_types=scratch)(kern)
    return fn(xl, s1, s2)


def _tc_combine(p, g1, g2, w1c, w2c, cntc, w_b, beta, plane, act):
    h = w_b.shape[1]
    hcat = p.shape[2]
    blk = 2000

    def kern(p_r, g1_r, g2_r, w1_r, w2_r, cnt_r, wb_r, beta_r, o_ref):
        out0 = p_r[plane, :, 0:h] + wb_r[...]
        num = (w1_r[...] * g1_r[:, 0:h] + w2_r[...] * g2_r[:, 0:h])
        out1 = num / jnp.maximum(cnt_r[...], 1.0)
        b = beta_r[0, 0]
        hh = b * out0 + (1.0 - b) * out1
        if act == "relu":
            o_ref[...] = jnp.maximum(hh, 0.0)
        else:
            m = jnp.max(hh, axis=1, keepdims=True)
            ex = jnp.exp(hh - m)
            o_ref[...] = (hh - m) - jnp.log(jnp.sum(ex, axis=1, keepdims=True))

    return pl.pallas_call(
        kern,
        grid=(N // blk,),
        in_specs=[
            pl.BlockSpec((2, blk, hcat), lambda i: (0, i, 0)),
            pl.BlockSpec((blk, 128), lambda i: (i, 0)),
            pl.BlockSpec((blk, 128), lambda i: (i, 0)),
            pl.BlockSpec((blk, 1), lambda i: (i, 0)),
            pl.BlockSpec((blk, 1), lambda i: (i, 0)),
            pl.BlockSpec((blk, 1), lambda i: (i, 0)),
            pl.BlockSpec((1, h), lambda i: (0, 0)),
            pl.BlockSpec((1, 1), lambda i: (0, 0)),
        ],
        out_specs=pl.BlockSpec((blk, h), lambda i: (i, 0)),
        out_shape=jax.ShapeDtypeStruct((N, h), jnp.float32),
    )(p, g1, g2, w1c, w2c, cntc, w_b, beta)


def kernel(x, edge_index, lin_w0, lin_b0, w_w0, w_b0, beta0,
           lin_w1, lin_b1, w_w1, w_b1, beta1):
    src = edge_index[0]
    dst = edge_index[1]
    h0 = lin_w0.shape[0]
    h1 = lin_w1.shape[0]

    xl0, nrm0 = _tc_linnorm(x, lin_w0, lin_b0)
    tab0, smin = _sc_sim_select(nrm0, src, dst, h0)
    b1v, b1e, b1s, b2v, b2e, b2s, cnt = _unpack_tab(tab0)
    w1, s1, w2, s2, cnt_m = _tc_merge(b1v, b1e, b1s, b2v, b2e, b2s, cnt)
    cntc = cnt_m.reshape(N, 1)

    row2d = _tc_rowidx(src.reshape(E // 128, 128), dst.reshape(E // 128, 128),
                       smin)
    w1tp = jnp.pad(w_w1.T, ((0, 0), (0, 128 - h1)))
    p = _sc_out0(w_w0.T, w1tp, dst, row2d.reshape(E))

    g1, g2 = _sc_gather_rows(xl0, s1.reshape(N), s2.reshape(N))
    hmid = _tc_combine(p, g1, g2, w1.reshape(N, 1), w2.reshape(N, 1), cntc,
                       w_b0.reshape(1, h0), beta0.reshape(1, 1), 0, "relu")

    xl1, nrm1 = _tc_linnorm(hmid, lin_w1, lin_b1)
    tab1, _ = _sc_sim_select(nrm1, src, dst, h1)
    c1v, c1e, c1s, c2v, c2e, c2s, _ = _unpack_tab(tab1)
    v1, t1, v2, t2, _ = _tc_merge(c1v, c1e, c1s, c2v, c2e, c2s, cnt)
    k1, k2 = _sc_gather_rows(xl1, t1.reshape(N), t2.reshape(N))
    out = _tc_combine(p, k1, k2, v1.reshape(N, 1), v2.reshape(N, 1), cntc,
                      w_b1.reshape(1, h1), beta1.reshape(1, 1),
                      1, "softmax")
    return out


# speedup vs baseline: 9.9645x; 1.1092x over previous
import functools
import jax
import jax.numpy as jnp
import numpy as np
from jax import lax
from jax.experimental import pallas as pl
from jax.experimental.pallas import tpu as pltpu
from jax.experimental.pallas import tpu_sc as plsc

N = 10000
E = 320000
NW = 32
EPW = E // NW
CH = 80
NCH = EPW // CH
SBE = 2000
SBC = SBE // CH
INT_MAX = np.int32(2**31 - 1)
NEG_INF = np.float32(-np.inf)
BIG_E = np.float32(2.0**30)
ACC_ROWS = 10112
DUMMY_ROW = N
ND_L = 336
ND_SP = 312


def _mesh():
    return plsc.VectorSubcoreMesh(core_axis_name="c", subcore_axis_name="s")


def _wid():
    return lax.axis_index("s") * 2 + lax.axis_index("c")


def _tc_linnorm(x, w, b):
    n, in_c = x.shape
    h = w.shape[0]
    blk = 1000

    def kern(x_ref, w_ref, b_ref, xl_ref, nrm_ref):
        xl = lax.dot_general(x_ref[...], w_ref[...], (((1,), (1,)), ((), ())),
                             preferred_element_type=jnp.float32) + b_ref[...]
        nn = jnp.sqrt(jnp.sum(xl * xl, axis=1, keepdims=True))
        nrm = xl / jnp.maximum(nn, 1e-12)
        if h < 128:
            z = jnp.zeros((blk, 128 - h), jnp.float32)
            xl = jnp.concatenate([xl, z], axis=1)
            nrm = jnp.concatenate([nrm, z], axis=1)
        xl_ref[...] = xl
        nrm_ref[...] = nrm

    return pl.pallas_call(
        kern,
        grid=(n // blk,),
        in_specs=[
            pl.BlockSpec((blk, in_c), lambda i: (i, 0)),
            pl.BlockSpec((h, in_c), lambda i: (0, 0)),
            pl.BlockSpec((1, h), lambda i: (0, 0)),
        ],
        out_specs=[
            pl.BlockSpec((blk, 128), lambda i: (i, 0)),
            pl.BlockSpec((blk, 128), lambda i: (i, 0)),
        ],
        out_shape=[
            jax.ShapeDtypeStruct((n, 128), jnp.float32),
            jax.ShapeDtypeStruct((n, 128), jnp.float32),
        ],
    )(x, w.reshape(h, in_c), b.reshape(1, h))


def _lanesum(v, redbuf):
    s1 = v + lax.rev(v, (0,))
    redbuf[pl.ds(0, 16)] = s1
    a = s1 + redbuf[pl.ds(4, 16)]
    redbuf[pl.ds(16, 16)] = a
    b = a + redbuf[pl.ds(18, 16)]
    redbuf[pl.ds(32, 16)] = b
    return (b + redbuf[pl.ds(33, 16)])[0]


TR = 5008


def _sc_sim_select(nrm, src, dst, flg, h):
    nsl = h // 16

    def kern(nrm_hbm, src_hbm, dst_hbm, flg_hbm, tab_o, smin_o,
             tab, sidx, didx, srowsA, drowsA, srowsB, drowsB,
             sminb, redbuf, flgbuf, semAs, semAd, semBs, semBd):
        wid = _wid()
        pltpu.sync_copy(flg_hbm.at[wid], flgbuf)
        lanes = lax.broadcasted_iota(jnp.int32, (16,), 0)
        l7 = lanes & 7
        initrow = jnp.where((l7 == 0) | (l7 == 3), NEG_INF,
                            jnp.where((l7 == 1) | (l7 == 4), BIG_E,
                                      0.0)).astype(jnp.float32)

        def init_body(i, carry):
            tab[pl.ds(i * 16, 16)] = initrow
            return carry

        lax.fori_loop(0, TR, init_body, 0)

        def issue(c, srows, drows, sems, semd):
            sl = pl.ds(c * CH, CH)
            pltpu.async_copy(nrm_hbm.at[sidx.at[sl]], srows, sems)
            pltpu.async_copy(nrm_hbm.at[didx.at[sl]], drows, semd)

        def wait(c, srows, drows, sems, semd):
            sl = pl.ds(c * CH, CH)
            pltpu.make_async_copy(nrm_hbm.at[sidx.at[sl]], srows, sems).wait()
            pltpu.make_async_copy(nrm_hbm.at[didx.at[sl]], drows, semd).wait()

        def upd_row(w0, cs, cand):
            c1v, c1e, c1s, c2v, c2e, c2s = cs
            vj, ej, sjf, kadd = cand
            gt1 = (vj > c1v) | ((vj == c1v) & (ej < c1e))
            gt2 = (vj > c2v) | ((vj == c2v) & (ej < c2e))
            n1v = jnp.where(gt1, vj, c1v)
            n1e = jnp.where(gt1, ej, c1e)
            n1s = jnp.where(gt1, sjf, c1s)
            n2v = jnp.where(gt1, c1v, jnp.where(gt2, vj, c2v))
            n2e = jnp.where(gt1, c1e, jnp.where(gt2, ej, c2e))
            n2s = jnp.where(gt1, c1s, jnp.where(gt2, sjf, c2s))
            nrow = jnp.where(lanes == 0, n1v, w0)
            nrow = jnp.where(lanes == 1, n1e, nrow)
            nrow = jnp.where(lanes == 2, n1s, nrow)
            nrow = jnp.where(lanes == 3, n2v, nrow)
            nrow = jnp.where(lanes == 4, n2e, nrow)
            nrow = jnp.where(lanes == 5, n2s, nrow)
            return jnp.where(lanes == 6, w0 + kadd, nrow)

        def compute(c, ebase, srows, drows, smin16):
            def group_body(g, smin16):
                s16 = sidx[pl.ds(c * CH + g * 16, 16)]
                d16 = didx[pl.ds(c * CH + g * 16, 16)]
                keep16 = s16 != d16
                smin16 = jnp.minimum(smin16, jnp.where(keep16, s16, N))
                cands = []
                ps = []
                for j in range(16):
                    r = g * 16 + j
                    acc = srows[r, pl.ds(0, 16)] * drows[r, pl.ds(0, 16)]
                    for s in range(1, nsl):
                        acc = acc + (srows[r, pl.ds(s * 16, 16)] *
                                     drows[r, pl.ds(s * 16, 16)])
                    sim_j = _lanesum(acc, redbuf)
                    dj = didx[pl.ds(c * CH + r, 16)][0]
                    sj = sidx[pl.ds(c * CH + r, 16)][0]
                    kj = sj != dj
                    vj = jnp.where(kj, sim_j, NEG_INF)
                    ej = jnp.where(kj, (ebase + c * CH + r)
                                   .astype(jnp.float32), BIG_E)
                    sjf = sj.astype(jnp.float32)
                    kadd = jnp.where(kj, 1.0, 0.0).astype(jnp.float32)
                    cands.append((vj, ej, sjf, kadd))
                    ps.append(dj * 8)

                gidx = (ebase - wid * EPW + c * CH) // 16 + g
                fl = flgbuf[pl.ds(gidx, 16)][0]

                def fields(p):
                    w0 = tab[pl.ds(p, 16)]
                    return w0, (w0[0],
                                tab[pl.ds(p + 1, 16)][0],
                                tab[pl.ds(p + 2, 16)][0],
                                tab[pl.ds(p + 3, 16)][0],
                                tab[pl.ds(p + 4, 16)][0],
                                tab[pl.ds(p + 5, 16)][0])

                @pl.when(fl == 0)
                def _():
                    for half in range(2):
                        loaded = [fields(ps[j]) for j in
                                  range(half * 8, half * 8 + 8)]
                        nrows = [upd_row(w0, cs, cands[half * 8 + i])
                                 for i, (w0, cs) in enumerate(loaded)]
                        for i, nr in enumerate(nrows):
                            tab[pl.ds(ps[half * 8 + i], 16)] = nr

                @pl.when(fl != 0)
                def _():
                    for j in range(16):
                        w0, cs = fields(ps[j])
                        tab[pl.ds(ps[j], 16)] = upd_row(w0, cs, cands[j])

                return smin16

            return lax.fori_loop(0, CH // 16, group_body, smin16)

        def sb_body(sb, smin16):
            ebase = wid * EPW + sb * SBE
            pltpu.sync_copy(src_hbm.at[pl.ds(ebase, SBE)],
                            sidx.at[pl.ds(0, SBE)])
            pltpu.sync_copy(dst_hbm.at[pl.ds(ebase, SBE)],
                            didx.at[pl.ds(0, SBE)])
            issue(0, srowsA, drowsA, semAs, semAd)

            def pair_body(i, smin16):
                wait(2 * i, srowsA, drowsA, semAs, semAd)
                issue(2 * i + 1, srowsB, drowsB, semBs, semBd)
                smin16 = compute(2 * i, ebase, srowsA, drowsA, smin16)
                wait(2 * i + 1, srowsB, drowsB, semBs, semBd)
                issue(2 * i + 2, srowsA, drowsA, semAs, semAd)
                return compute(2 * i + 1, ebase, srowsB, drowsB, smin16)

            smin16 = lax.fori_loop(0, SBC // 2, pair_body, smin16)
            wait(SBC - 1, srowsA, drowsA, semAs, semAd)
            return compute(SBC - 1, ebase, srowsA, drowsA, smin16)

        smin16 = lax.fori_loop(0, EPW // SBE, sb_body,
                               jnp.full((16,), N, jnp.int32))
        sminb[...] = smin16
        pltpu.sync_copy(tab.at[pl.ds(0, TR * 16)], tab_o.at[wid])
        pltpu.sync_copy(sminb, smin_o.at[wid])

    f32 = jnp.float32
    i32 = jnp.int32
    out_type = [
        jax.ShapeDtypeStruct((NW, TR * 16), f32),
        jax.ShapeDtypeStruct((NW, 16), i32),
    ]
    scratch = [
        pltpu.VMEM((TR * 16 + 16,), f32),
        pltpu.VMEM((SBE + 16,), i32), pltpu.VMEM((SBE + 16,), i32),
        pltpu.VMEM((CH, 128), f32), pltpu.VMEM((CH, 128), f32),
        pltpu.VMEM((CH, 128), f32), pltpu.VMEM((CH, 128), f32),
        pltpu.VMEM((16,), i32),
        pltpu.VMEM((64,), f32),
        pltpu.VMEM((640,), i32),
        pltpu.SemaphoreType.DMA, pltpu.SemaphoreType.DMA,
        pltpu.SemaphoreType.DMA, pltpu.SemaphoreType.DMA,
    ]
    fn = functools.partial(pl.kernel, mesh=_mesh(), out_type=out_type,
                           scratch_types=scratch)(kern)
    return fn(nrm, src, dst, flg)


def _unpack_tab(tab):
    t = tab.reshape(NW, TR * 2, 8)[:, :N, :]
    return (t[:, :, 0], t[:, :, 1], t[:, :, 2],
            t[:, :, 3], t[:, :, 4], t[:, :, 5], t[:, :, 6])


def _tc_merge(b1v, b1e, b1s, b2v, b2e, b2s, cnt):
    blk = N

    def kern(b1v_r, b1e_r, b1s_r, b2v_r, b2e_r, b2s_r, cnt_r,
             w1_o, s1_o, w2_o, s2_o, cnt_o):
        a1v = b1v_r[pl.ds(0, 1), :]
        a1e = b1e_r[pl.ds(0, 1), :]
        a1s = b1s_r[pl.ds(0, 1), :]
        a2v = b2v_r[pl.ds(0, 1), :]
        a2e = b2e_r[pl.ds(0, 1), :]
        a2s = b2s_r[pl.ds(0, 1), :]

        def body(j, carry):
            a1v, a1e, a1s, a2v, a2e, a2s = carry
            c1v = b1v_r[pl.ds(j, 1), :]
            c1e = b1e_r[pl.ds(j, 1), :]
            c1s = b1s_r[pl.ds(j, 1), :]
            c2v = b2v_r[pl.ds(j, 1), :]
            c2e = b2e_r[pl.ds(j, 1), :]
            c2s = b2s_r[pl.ds(j, 1), :]
            gA = (a1v > c1v) | ((a1v == c1v) & (a1e < c1e))
            n1v = jnp.where(gA, a1v, c1v)
            n1e = jnp.where(gA, a1e, c1e)
            n1s = jnp.where(gA, a1s, c1s)
            l1v = jnp.where(gA, c1v, a1v)
            l1e = jnp.where(gA, c1e, a1e)
            l1s = jnp.where(gA, c1s, a1s)
            p2v = jnp.where(gA, a2v, c2v)
            p2e = jnp.where(gA, a2e, c2e)
            p2s = jnp.where(gA, a2s, c2s)
            gB = (l1v > p2v) | ((l1v == p2v) & (l1e < p2e))
            n2v = jnp.where(gB, l1v, p2v)
            n2e = jnp.where(gB, l1e, p2e)
            n2s = jnp.where(gB, l1s, p2s)
            return (n1v, n1e, n1s, n2v, n2e, n2s)

        a1v, a1e, a1s, a2v, a2e, a2s = lax.fori_loop(
            1, NW, body, (a1v, a1e, a1s, a2v, a2e, a2s))
        val1 = (a1e < BIG_E) & (a1v >= 0.0)
        val2 = (a2e < BIG_E) & (a2v >= 0.0)
        w1_o[...] = jnp.where(val1, a1v, 0.0)
        s1_o[...] = jnp.where(val1, a1s, 0.0).astype(jnp.int32)
        w2_o[...] = jnp.where(val2, a2v, 0.0)
        s2_o[...] = jnp.where(val2, a2s, 0.0).astype(jnp.int32)
        cnt_o[...] = jnp.sum(cnt_r[...], axis=0, keepdims=True)

    f32 = jnp.float32
    i32 = jnp.int32
    io = pl.BlockSpec((NW, blk), lambda i: (0, 0))
    oo = pl.BlockSpec((1, blk), lambda i: (0, 0))
    return pl.pallas_call(
        kern,
        grid=(N // blk,),
        in_specs=[io] * 7,
        out_specs=[oo] * 5,
        out_shape=[
            jax.ShapeDtypeStruct((1, N), f32),
            jax.ShapeDtypeStruct((1, N), i32),
            jax.ShapeDtypeStruct((1, N), f32),
            jax.ShapeDtypeStruct((1, N), i32),
            jax.ShapeDtypeStruct((1, N), f32),
        ],
    )(b1v, b1e, b1s, b2v, b2e, b2s, cnt)


def _tc_rowidx(src2d, dst2d, smin):
    blk = src2d.shape[0]

    def kern(src_r, dst_r, smin_r, row_o):
        sm = jnp.min(smin_r[...])
        s = src_r[...]
        d = dst_r[...]
        row_o[...] = jnp.where(s != d, s - sm, DUMMY_ROW)

    return pl.pallas_call(
        kern,
        grid=(src2d.shape[0] // blk,),
        in_specs=[
            pl.BlockSpec((blk, 128), lambda i: (i, 0)),
            pl.BlockSpec((blk, 128), lambda i: (i, 0)),
            pl.BlockSpec((NW, 16), lambda i: (0, 0)),
        ],
        out_specs=pl.BlockSpec((blk, 128), lambda i: (i, 0)),
        out_shape=jax.ShapeDtypeStruct(src2d.shape, jnp.int32),
    )(src2d, dst2d, smin)


def _sc_out0(w0t, w1tp, dst, row):
    hp = 128
    zrows = 104
    rows_per_tile = ACC_ROWS // 16
    epw = E // 16
    CHO = 80
    SBO = 2000
    SBOC = SBO // CHO

    def kern(w0_hbm, w1_hbm, dst_hbm, row_hbm, out_hbm,
             didx, ridxA, ridxB, wrowsA, wrowsB, zbuf, acc,
             semAg, semAr, semBg, semBr):
        cid = lax.axis_index("c")
        sid = lax.axis_index("s")

        def zinit(i, carry):
            r = i // (hp // 16)
            cc = (i % (hp // 16)) * 16
            zbuf[r, pl.ds(cc, 16)] = jnp.zeros((16,), jnp.float32)
            return carry

        lax.fori_loop(0, zrows * (hp // 16), zinit, 0)
        zb = sid * rows_per_tile
        for t in range(6):
            pltpu.sync_copy(zbuf, acc.at[pl.ds(zb + t * zrows, zrows)])
        pltpu.sync_copy(zbuf.at[pl.ds(0, 8)],
                        acc.at[pl.ds(zb + 6 * zrows, 8)])
        plsc.subcore_barrier()

        def make_body(w_hbm):
            def issue(sbase, c, wrows, ridx, semg, semr):
                sl = pl.ds(c * CHO, CHO)
                pltpu.async_copy(w_hbm.at[didx.at[sl]], wrows, semg)
                pltpu.async_copy(row_hbm.at[pl.ds(sbase + c * CHO, CHO)],
                                 ridx, semr)

            def wait(sbase, c, wrows, ridx, semg, semr):
                sl = pl.ds(c * CHO, CHO)
                pltpu.make_async_copy(w_hbm.at[didx.at[sl]],
                                      wrows, semg).wait()
                pltpu.make_async_copy(row_hbm.at[pl.ds(sbase + c * CHO, CHO)],
                                      ridx, semr).wait()

            def scat(wrows, ridx):
                pltpu.sync_copy(wrows, acc.at[ridx], add=True)

            def sb_body(sb, carry):
                sbase = sid * epw + sb * SBO
                pltpu.sync_copy(dst_hbm.at[pl.ds(sbase, SBO)], didx)
                issue(sbase, 0, wrowsA, ridxA, semAg, semAr)

                def pair_body(i, carry):
                    wait(sbase, 2 * i, wrowsA, ridxA, semAg, semAr)
                    issue(sbase, 2 * i + 1, wrowsB, ridxB, semBg, semBr)
                    scat(wrowsA, ridxA)
                    wait(sbase, 2 * i + 1, wrowsB, ridxB, semBg, semBr)
                    issue(sbase, 2 * i + 2, wrowsA, ridxA, semAg, semAr)
                    scat(wrowsB, ridxB)
                    return carry

                lax.fori_loop(0, SBOC // 2, pair_body, 0)
                wait(sbase, SBOC - 1, wrowsA, ridxA, semAg, semAr)
                scat(wrowsA, ridxA)
                return carry

            lax.fori_loop(0, epw // SBO, sb_body, 0)

        @pl.when(cid == 0)
        def _():
            make_body(w0_hbm)

        @pl.when(cid == 1)
        def _():
            make_body(w1_hbm)

        plsc.subcore_barrier()

        @pl.when(sid == 0)
        def _():
            pltpu.sync_copy(acc, out_hbm.at[cid])

    out_type = jax.ShapeDtypeStruct((2, ACC_ROWS, hp), jnp.float32)
    scratch = [
        pltpu.VMEM((SBO,), jnp.int32),
        pltpu.VMEM((CHO,), jnp.int32), pltpu.VMEM((CHO,), jnp.int32),
        pltpu.VMEM((CHO, hp), jnp.float32), pltpu.VMEM((CHO, hp), jnp.float32),
        pltpu.VMEM((zrows, hp), jnp.float32),
        pltpu.VMEM_SHARED((ACC_ROWS, hp), jnp.float32),
        pltpu.SemaphoreType.DMA, pltpu.SemaphoreType.DMA,
        pltpu.SemaphoreType.DMA, pltpu.SemaphoreType.DMA,
    ]
    fn = functools.partial(pl.kernel, mesh=_mesh(), out_type=out_type,
                           scratch_types=scratch)(kern)
    return fn(w0t, w1tp, dst, row)


def _sc_gather_rows(xl, s1, s2):
    def kern(xl_hbm, s1_hbm, s2_hbm, g1_o, g2_o,
             i1, i2, r1, r2, sem1, sem2):
        wid = _wid()
        base = jnp.where(wid == NW - 1, N - ND_L, wid * ND_SP)
        pltpu.sync_copy(s1_hbm.at[pl.ds(base, ND_L)], i1)
        pltpu.sync_copy(s2_hbm.at[pl.ds(base, ND_L)], i2)
        cp1 = pltpu.async_copy(xl_hbm.at[i1], r1, sem1)
        cp2 = pltpu.async_copy(xl_hbm.at[i2], r2, sem2)
        cp1.wait()
        cp2.wait()
        pltpu.sync_copy(r1, g1_o.at[pl.ds(base, ND_L)])
        pltpu.sync_copy(r2, g2_o.at[pl.ds(base, ND_L)])

    out_type = [
        jax.ShapeDtypeStruct((N, 128), jnp.float32),
        jax.ShapeDtypeStruct((N, 128), jnp.float32),
    ]
    scratch = [
        pltpu.VMEM((ND_L,), jnp.int32), pltpu.VMEM((ND_L,), jnp.int32),
        pltpu.VMEM((ND_L, 128), jnp.float32),
        pltpu.VMEM((ND_L, 128), jnp.float32),
        pltpu.SemaphoreType.DMA, pltpu.SemaphoreType.DMA,
    ]
    fn = functools.partial(pl.kernel, mesh=_mesh(), out_type=out_type,
                           scratch_types=scratch)(kern)
    return fn(xl, s1, s2)


def _tc_combine(p, g1, g2, w1c, w2c, cntc, w_b, beta, plane, act):
    h = w_b.shape[1]
    hcat = p.shape[2]
    blk = 2000

    def kern(p_r, g1_r, g2_r, w1_r, w2_r, cnt_r, wb_r, beta_r, o_ref):
        out0 = p_r[plane, :, 0:h] + wb_r[...]
        num = (w1_r[...] * g1_r[:, 0:h] + w2_r[...] * g2_r[:, 0:h])
        out1 = num / jnp.maximum(cnt_r[...], 1.0)
        b = beta_r[0, 0]
        hh = b * out0 + (1.0 - b) * out1
        if act == "relu":
            o_ref[...] = jnp.maximum(hh, 0.0)
        else:
            m = jnp.max(hh, axis=1, keepdims=True)
            ex = jnp.exp(hh - m)
            o_ref[...] = (hh - m) - jnp.log(jnp.sum(ex, axis=1, keepdims=True))

    return pl.pallas_call(
        kern,
        grid=(N // blk,),
        in_specs=[
            pl.BlockSpec((2, blk, hcat), lambda i: (0, i, 0)),
            pl.BlockSpec((blk, 128), lambda i: (i, 0)),
            pl.BlockSpec((blk, 128), lambda i: (i, 0)),
            pl.BlockSpec((blk, 1), lambda i: (i, 0)),
            pl.BlockSpec((blk, 1), lambda i: (i, 0)),
            pl.BlockSpec((blk, 1), lambda i: (i, 0)),
            pl.BlockSpec((1, h), lambda i: (0, 0)),
            pl.BlockSpec((1, 1), lambda i: (0, 0)),
        ],
        out_specs=pl.BlockSpec((blk, h), lambda i: (i, 0)),
        out_shape=jax.ShapeDtypeStruct((N, h), jnp.float32),
    )(p, g1, g2, w1c, w2c, cntc, w_b, beta)


def kernel(x, edge_index, lin_w0, lin_b0, w_w0, w_b0, beta0,
           lin_w1, lin_b1, w_w1, w_b1, beta1):
    src = edge_index[0]
    dst = edge_index[1]
    h0 = lin_w0.shape[0]
    h1 = lin_w1.shape[0]

    dg = dst.reshape(E // 16, 16)
    dd = jnp.abs(dg[:, :, None] - dg[:, None, :]) <= 1
    eye = jnp.eye(16, dtype=bool)
    conf = jnp.any(dd & ~eye[None], axis=(1, 2)).astype(jnp.int32)
    flg = jnp.pad(conf.reshape(NW, EPW // 16), ((0, 0), (0, 640 - EPW // 16)))

    xl0, nrm0 = _tc_linnorm(x, lin_w0, lin_b0)
    tab0, smin = _sc_sim_select(nrm0, src, dst, flg, h0)
    b1v, b1e, b1s, b2v, b2e, b2s, cnt = _unpack_tab(tab0)
    w1, s1, w2, s2, cnt_m = _tc_merge(b1v, b1e, b1s, b2v, b2e, b2s, cnt)
    cntc = cnt_m.reshape(N, 1)

    row2d = _tc_rowidx(src.reshape(E // 128, 128), dst.reshape(E // 128, 128),
                       smin)
    w1tp = jnp.pad(w_w1.T, ((0, 0), (0, 128 - h1)))
    p = _sc_out0(w_w0.T, w1tp, dst, row2d.reshape(E))

    g1, g2 = _sc_gather_rows(xl0, s1.reshape(N), s2.reshape(N))
    hmid = _tc_combine(p, g1, g2, w1.reshape(N, 1), w2.reshape(N, 1), cntc,
                       w_b0.reshape(1, h0), beta0.reshape(1, 1), 0, "relu")

    xl1, nrm1 = _tc_linnorm(hmid, lin_w1, lin_b1)
    tab1, _ = _sc_sim_select(nrm1, src, dst, flg, h1)
    c1v, c1e, c1s, c2v, c2e, c2s, _ = _unpack_tab(tab1)
    v1, t1, v2, t2, _ = _tc_merge(c1v, c1e, c1s, c2v, c2e, c2s, cnt)
    k1, k2 = _sc_gather_rows(xl1, t1.reshape(N), t2.reshape(N))
    out = _tc_combine(p, k1, k2, v1.reshape(N, 1), v2.reshape(N, 1), cntc,
                      w_b1.reshape(1, h1), beta1.reshape(1, 1),
                      1, "softmax")
    return out


# speedup vs baseline: 10.2937x; 1.0330x over previous
import functools
import jax
import jax.numpy as jnp
import numpy as np
from jax import lax
from jax.experimental import pallas as pl
from jax.experimental.pallas import tpu as pltpu
from jax.experimental.pallas import tpu_sc as plsc

N = 10000
E = 320000
NW = 32
EPW = E // NW
CH = 80
NCH = EPW // CH
SBE = 2000
SBC = SBE // CH
INT_MAX = np.int32(2**31 - 1)
NEG_INF = np.float32(-np.inf)
BIG_E = np.float32(2.0**30)
ACC_ROWS = 10112
DUMMY_ROW = N
ND_L = 336
ND_SP = 312


def _mesh():
    return plsc.VectorSubcoreMesh(core_axis_name="c", subcore_axis_name="s")


def _wid():
    return lax.axis_index("s") * 2 + lax.axis_index("c")


def _tc_linnorm(x, w, b):
    n, in_c = x.shape
    h = w.shape[0]
    blk = 1000

    def kern(x_ref, w_ref, b_ref, xl_ref, nrm_ref):
        xl = lax.dot_general(x_ref[...], w_ref[...], (((1,), (1,)), ((), ())),
                             preferred_element_type=jnp.float32) + b_ref[...]
        nn = jnp.sqrt(jnp.sum(xl * xl, axis=1, keepdims=True))
        nrm = xl / jnp.maximum(nn, 1e-12)
        if h < 128:
            z = jnp.zeros((blk, 128 - h), jnp.float32)
            xl = jnp.concatenate([xl, z], axis=1)
            nrm = jnp.concatenate([nrm, z], axis=1)
        xl_ref[...] = xl
        nrm_ref[...] = nrm

    return pl.pallas_call(
        kern,
        grid=(n // blk,),
        in_specs=[
            pl.BlockSpec((blk, in_c), lambda i: (i, 0)),
            pl.BlockSpec((h, in_c), lambda i: (0, 0)),
            pl.BlockSpec((1, h), lambda i: (0, 0)),
        ],
        out_specs=[
            pl.BlockSpec((blk, 128), lambda i: (i, 0)),
            pl.BlockSpec((blk, 128), lambda i: (i, 0)),
        ],
        out_shape=[
            jax.ShapeDtypeStruct((n, 128), jnp.float32),
            jax.ShapeDtypeStruct((n, 128), jnp.float32),
        ],
    )(x, w.reshape(h, in_c), b.reshape(1, h))


def _lanesum(v, redbuf, slot):
    o = slot * 32
    s1 = v + lax.rev(v, (0,))
    redbuf[pl.ds(o, 16)] = s1
    w1 = redbuf[pl.ds(o + 1, 16)]
    w2 = redbuf[pl.ds(o + 2, 16)]
    w3 = redbuf[pl.ds(o + 3, 16)]
    w4 = redbuf[pl.ds(o + 4, 16)]
    w5 = redbuf[pl.ds(o + 5, 16)]
    w6 = redbuf[pl.ds(o + 6, 16)]
    w7 = redbuf[pl.ds(o + 7, 16)]
    t = ((s1 + w1) + (w2 + w3)) + ((w4 + w5) + (w6 + w7))
    return t[0]


TR = 5008


def _sc_sim_select(nrm, src, dst, flg, h):
    nsl = h // 16

    def kern(nrm_hbm, src_hbm, dst_hbm, flg_hbm, tab_o, smin_o,
             tab, sidx, didx, srowsA, drowsA, srowsB, drowsB,
             sminb, redbuf, flgbuf, semAs, semAd, semBs, semBd):
        wid = _wid()
        pltpu.sync_copy(flg_hbm.at[wid], flgbuf)
        lanes = lax.broadcasted_iota(jnp.int32, (16,), 0)
        l7 = lanes & 7
        initrow = jnp.where((l7 == 0) | (l7 == 3), NEG_INF,
                            jnp.where((l7 == 1) | (l7 == 4), BIG_E,
                                      0.0)).astype(jnp.float32)

        def init_body(i, carry):
            tab[pl.ds(i * 16, 16)] = initrow
            return carry

        lax.fori_loop(0, TR, init_body, 0)

        def issue(c, srows, drows, sems, semd):
            sl = pl.ds(c * CH, CH)
            pltpu.async_copy(nrm_hbm.at[sidx.at[sl]], srows, sems)
            pltpu.async_copy(nrm_hbm.at[didx.at[sl]], drows, semd)

        def wait(c, srows, drows, sems, semd):
            sl = pl.ds(c * CH, CH)
            pltpu.make_async_copy(nrm_hbm.at[sidx.at[sl]], srows, sems).wait()
            pltpu.make_async_copy(nrm_hbm.at[didx.at[sl]], drows, semd).wait()

        def upd_row(w0, cs, cand):
            c1v, c1e, c1s, c2v, c2e, c2s = cs
            vj, ej, sjf, kadd = cand
            gt1 = (vj > c1v) | ((vj == c1v) & (ej < c1e))
            gt2 = (vj > c2v) | ((vj == c2v) & (ej < c2e))
            n1v = jnp.where(gt1, vj, c1v)
            n1e = jnp.where(gt1, ej, c1e)
            n1s = jnp.where(gt1, sjf, c1s)
            n2v = jnp.where(gt1, c1v, jnp.where(gt2, vj, c2v))
            n2e = jnp.where(gt1, c1e, jnp.where(gt2, ej, c2e))
            n2s = jnp.where(gt1, c1s, jnp.where(gt2, sjf, c2s))
            nrow = jnp.where(lanes == 0, n1v, w0)
            nrow = jnp.where(lanes == 1, n1e, nrow)
            nrow = jnp.where(lanes == 2, n1s, nrow)
            nrow = jnp.where(lanes == 3, n2v, nrow)
            nrow = jnp.where(lanes == 4, n2e, nrow)
            nrow = jnp.where(lanes == 5, n2s, nrow)
            return jnp.where(lanes == 6, w0 + kadd, nrow)

        def compute(c, ebase, srows, drows, smin16):
            def group_body(g, smin16):
                s16 = sidx[pl.ds(c * CH + g * 16, 16)]
                d16 = didx[pl.ds(c * CH + g * 16, 16)]
                keep16 = s16 != d16
                smin16 = jnp.minimum(smin16, jnp.where(keep16, s16, N))
                cands = []
                ps = []
                for j in range(16):
                    r = g * 16 + j
                    acc = srows[r, pl.ds(0, 16)] * drows[r, pl.ds(0, 16)]
                    for s in range(1, nsl):
                        acc = acc + (srows[r, pl.ds(s * 16, 16)] *
                                     drows[r, pl.ds(s * 16, 16)])
                    sim_j = _lanesum(acc, redbuf, j)
                    dj = didx[pl.ds(c * CH + r, 16)][0]
                    sj = sidx[pl.ds(c * CH + r, 16)][0]
                    kj = sj != dj
                    vj = jnp.where(kj, sim_j, NEG_INF)
                    ej = jnp.where(kj, (ebase + c * CH + r)
                                   .astype(jnp.float32), BIG_E)
                    sjf = sj.astype(jnp.float32)
                    kadd = jnp.where(kj, 1.0, 0.0).astype(jnp.float32)
                    cands.append((vj, ej, sjf, kadd))
                    ps.append(dj * 8)

                gidx = (ebase - wid * EPW + c * CH) // 16 + g
                fl = flgbuf[pl.ds(gidx, 16)][0]

                def fields(p):
                    w0 = tab[pl.ds(p, 16)]
                    return w0, (w0[0],
                                tab[pl.ds(p + 1, 16)][0],
                                tab[pl.ds(p + 2, 16)][0],
                                tab[pl.ds(p + 3, 16)][0],
                                tab[pl.ds(p + 4, 16)][0],
                                tab[pl.ds(p + 5, 16)][0])

                @pl.when(fl == 0)
                def _():
                    for half in range(2):
                        loaded = [fields(ps[j]) for j in
                                  range(half * 8, half * 8 + 8)]
                        nrows = [upd_row(w0, cs, cands[half * 8 + i])
                                 for i, (w0, cs) in enumerate(loaded)]
                        for i, nr in enumerate(nrows):
                            tab[pl.ds(ps[half * 8 + i], 16)] = nr

                @pl.when(fl != 0)
                def _():
                    for j in range(16):
                        w0, cs = fields(ps[j])
                        tab[pl.ds(ps[j], 16)] = upd_row(w0, cs, cands[j])

                return smin16

            return lax.fori_loop(0, CH // 16, group_body, smin16)

        def sb_body(sb, smin16):
            ebase = wid * EPW + sb * SBE
            pltpu.sync_copy(src_hbm.at[pl.ds(ebase, SBE)],
                            sidx.at[pl.ds(0, SBE)])
            pltpu.sync_copy(dst_hbm.at[pl.ds(ebase, SBE)],
                            didx.at[pl.ds(0, SBE)])
            issue(0, srowsA, drowsA, semAs, semAd)

            def pair_body(i, smin16):
                wait(2 * i, srowsA, drowsA, semAs, semAd)
                issue(2 * i + 1, srowsB, drowsB, semBs, semBd)
                smin16 = compute(2 * i, ebase, srowsA, drowsA, smin16)
                wait(2 * i + 1, srowsB, drowsB, semBs, semBd)
                issue(2 * i + 2, srowsA, drowsA, semAs, semAd)
                return compute(2 * i + 1, ebase, srowsB, drowsB, smin16)

            smin16 = lax.fori_loop(0, SBC // 2, pair_body, smin16)
            wait(SBC - 1, srowsA, drowsA, semAs, semAd)
            return compute(SBC - 1, ebase, srowsA, drowsA, smin16)

        smin16 = lax.fori_loop(0, EPW // SBE, sb_body,
                               jnp.full((16,), N, jnp.int32))
        sminb[...] = smin16
        pltpu.sync_copy(tab.at[pl.ds(0, TR * 16)], tab_o.at[wid])
        pltpu.sync_copy(sminb, smin_o.at[wid])

    f32 = jnp.float32
    i32 = jnp.int32
    out_type = [
        jax.ShapeDtypeStruct((NW, TR * 16), f32),
        jax.ShapeDtypeStruct((NW, 16), i32),
    ]
    scratch = [
        pltpu.VMEM((TR * 16 + 16,), f32),
        pltpu.VMEM((SBE + 16,), i32), pltpu.VMEM((SBE + 16,), i32),
        pltpu.VMEM((CH, 128), f32), pltpu.VMEM((CH, 128), f32),
        pltpu.VMEM((CH, 128), f32), pltpu.VMEM((CH, 128), f32),
        pltpu.VMEM((16,), i32),
        pltpu.VMEM((528,), f32),
        pltpu.VMEM((640,), i32),
        pltpu.SemaphoreType.DMA, pltpu.SemaphoreType.DMA,
        pltpu.SemaphoreType.DMA, pltpu.SemaphoreType.DMA,
    ]
    fn = functools.partial(pl.kernel, mesh=_mesh(), out_type=out_type,
                           scratch_types=scratch)(kern)
    return fn(nrm, src, dst, flg)


def _unpack_tab(tab):
    t = tab.reshape(NW, TR * 2, 8)[:, :N, :]
    return (t[:, :, 0], t[:, :, 1], t[:, :, 2],
            t[:, :, 3], t[:, :, 4], t[:, :, 5], t[:, :, 6])


def _tc_merge(b1v, b1e, b1s, b2v, b2e, b2s, cnt):
    blk = N

    def kern(b1v_r, b1e_r, b1s_r, b2v_r, b2e_r, b2s_r, cnt_r,
             w1_o, s1_o, w2_o, s2_o, cnt_o):
        a1v = b1v_r[pl.ds(0, 1), :]
        a1e = b1e_r[pl.ds(0, 1), :]
        a1s = b1s_r[pl.ds(0, 1), :]
        a2v = b2v_r[pl.ds(0, 1), :]
        a2e = b2e_r[pl.ds(0, 1), :]
        a2s = b2s_r[pl.ds(0, 1), :]

        def body(j, carry):
            a1v, a1e, a1s, a2v, a2e, a2s = carry
            c1v = b1v_r[pl.ds(j, 1), :]
            c1e = b1e_r[pl.ds(j, 1), :]
            c1s = b1s_r[pl.ds(j, 1), :]
            c2v = b2v_r[pl.ds(j, 1), :]
            c2e = b2e_r[pl.ds(j, 1), :]
            c2s = b2s_r[pl.ds(j, 1), :]
            gA = (a1v > c1v) | ((a1v == c1v) & (a1e < c1e))
            n1v = jnp.where(gA, a1v, c1v)
            n1e = jnp.where(gA, a1e, c1e)
            n1s = jnp.where(gA, a1s, c1s)
            l1v = jnp.where(gA, c1v, a1v)
            l1e = jnp.where(gA, c1e, a1e)
            l1s = jnp.where(gA, c1s, a1s)
            p2v = jnp.where(gA, a2v, c2v)
            p2e = jnp.where(gA, a2e, c2e)
            p2s = jnp.where(gA, a2s, c2s)
            gB = (l1v > p2v) | ((l1v == p2v) & (l1e < p2e))
            n2v = jnp.where(gB, l1v, p2v)
            n2e = jnp.where(gB, l1e, p2e)
            n2s = jnp.where(gB, l1s, p2s)
            return (n1v, n1e, n1s, n2v, n2e, n2s)

        a1v, a1e, a1s, a2v, a2e, a2s = lax.fori_loop(
            1, NW, body, (a1v, a1e, a1s, a2v, a2e, a2s))
        val1 = (a1e < BIG_E) & (a1v >= 0.0)
        val2 = (a2e < BIG_E) & (a2v >= 0.0)
        w1_o[...] = jnp.where(val1, a1v, 0.0)
        s1_o[...] = jnp.where(val1, a1s, 0.0).astype(jnp.int32)
        w2_o[...] = jnp.where(val2, a2v, 0.0)
        s2_o[...] = jnp.where(val2, a2s, 0.0).astype(jnp.int32)
        cnt_o[...] = jnp.sum(cnt_r[...], axis=0, keepdims=True)

    f32 = jnp.float32
    i32 = jnp.int32
    io = pl.BlockSpec((NW, blk), lambda i: (0, 0))
    oo = pl.BlockSpec((1, blk), lambda i: (0, 0))
    return pl.pallas_call(
        kern,
        grid=(N // blk,),
        in_specs=[io] * 7,
        out_specs=[oo] * 5,
        out_shape=[
            jax.ShapeDtypeStruct((1, N), f32),
            jax.ShapeDtypeStruct((1, N), i32),
            jax.ShapeDtypeStruct((1, N), f32),
            jax.ShapeDtypeStruct((1, N), i32),
            jax.ShapeDtypeStruct((1, N), f32),
        ],
    )(b1v, b1e, b1s, b2v, b2e, b2s, cnt)


def _tc_rowidx(src2d, dst2d, smin):
    blk = src2d.shape[0]

    def kern(src_r, dst_r, smin_r, row_o):
        sm = jnp.min(smin_r[...])
        s = src_r[...]
        d = dst_r[...]
        row_o[...] = jnp.where(s != d, s - sm, DUMMY_ROW)

    return pl.pallas_call(
        kern,
        grid=(src2d.shape[0] // blk,),
        in_specs=[
            pl.BlockSpec((blk, 128), lambda i: (i, 0)),
            pl.BlockSpec((blk, 128), lambda i: (i, 0)),
            pl.BlockSpec((NW, 16), lambda i: (0, 0)),
        ],
        out_specs=pl.BlockSpec((blk, 128), lambda i: (i, 0)),
        out_shape=jax.ShapeDtypeStruct(src2d.shape, jnp.int32),
    )(src2d, dst2d, smin)


def _sc_out0(w0t, w1tp, dst, row):
    hp = 128
    zrows = 104
    rows_per_tile = ACC_ROWS // 16
    epw = E // 16
    CHO = 80
    SBO = 2000
    SBOC = SBO // CHO

    def kern(w0_hbm, w1_hbm, dst_hbm, row_hbm, out_hbm,
             didx, ridxA, ridxB, wrowsA, wrowsB, zbuf, acc,
             semAg, semAr, semBg, semBr):
        cid = lax.axis_index("c")
        sid = lax.axis_index("s")

        def zinit(i, carry):
            r = i // (hp // 16)
            cc = (i % (hp // 16)) * 16
            zbuf[r, pl.ds(cc, 16)] = jnp.zeros((16,), jnp.float32)
            return carry

        lax.fori_loop(0, zrows * (hp // 16), zinit, 0)
        zb = sid * rows_per_tile
        for t in range(6):
            pltpu.sync_copy(zbuf, acc.at[pl.ds(zb + t * zrows, zrows)])
        pltpu.sync_copy(zbuf.at[pl.ds(0, 8)],
                        acc.at[pl.ds(zb + 6 * zrows, 8)])
        plsc.subcore_barrier()

        def make_body(w_hbm):
            def issue(sbase, c, wrows, ridx, semg, semr):
                sl = pl.ds(c * CHO, CHO)
                pltpu.async_copy(w_hbm.at[didx.at[sl]], wrows, semg)
                pltpu.async_copy(row_hbm.at[pl.ds(sbase + c * CHO, CHO)],
                                 ridx, semr)

            def wait(sbase, c, wrows, ridx, semg, semr):
                sl = pl.ds(c * CHO, CHO)
                pltpu.make_async_copy(w_hbm.at[didx.at[sl]],
                                      wrows, semg).wait()
                pltpu.make_async_copy(row_hbm.at[pl.ds(sbase + c * CHO, CHO)],
                                      ridx, semr).wait()

            def scat(wrows, ridx):
                pltpu.sync_copy(wrows, acc.at[ridx], add=True)

            def sb_body(sb, carry):
                sbase = sid * epw + sb * SBO
                pltpu.sync_copy(dst_hbm.at[pl.ds(sbase, SBO)], didx)
                issue(sbase, 0, wrowsA, ridxA, semAg, semAr)

                def pair_body(i, carry):
                    wait(sbase, 2 * i, wrowsA, ridxA, semAg, semAr)
                    issue(sbase, 2 * i + 1, wrowsB, ridxB, semBg, semBr)
                    scat(wrowsA, ridxA)
                    wait(sbase, 2 * i + 1, wrowsB, ridxB, semBg, semBr)
                    issue(sbase, 2 * i + 2, wrowsA, ridxA, semAg, semAr)
                    scat(wrowsB, ridxB)
                    return carry

                lax.fori_loop(0, SBOC // 2, pair_body, 0)
                wait(sbase, SBOC - 1, wrowsA, ridxA, semAg, semAr)
                scat(wrowsA, ridxA)
                return carry

            lax.fori_loop(0, epw // SBO, sb_body, 0)

        @pl.when(cid == 0)
        def _():
            make_body(w0_hbm)

        @pl.when(cid == 1)
        def _():
            make_body(w1_hbm)

        plsc.subcore_barrier()

        @pl.when(sid == 0)
        def _():
            pltpu.sync_copy(acc, out_hbm.at[cid])

    out_type = jax.ShapeDtypeStruct((2, ACC_ROWS, hp), jnp.float32)
    scratch = [
        pltpu.VMEM((SBO,), jnp.int32),
        pltpu.VMEM((CHO,), jnp.int32), pltpu.VMEM((CHO,), jnp.int32),
        pltpu.VMEM((CHO, hp), jnp.float32), pltpu.VMEM((CHO, hp), jnp.float32),
        pltpu.VMEM((zrows, hp), jnp.float32),
        pltpu.VMEM_SHARED((ACC_ROWS, hp), jnp.float32),
        pltpu.SemaphoreType.DMA, pltpu.SemaphoreType.DMA,
        pltpu.SemaphoreType.DMA, pltpu.SemaphoreType.DMA,
    ]
    fn = functools.partial(pl.kernel, mesh=_mesh(), out_type=out_type,
                           scratch_types=scratch)(kern)
    return fn(w0t, w1tp, dst, row)


def _sc_gather_rows(xl, s1, s2):
    def kern(xl_hbm, s1_hbm, s2_hbm, g1_o, g2_o,
             i1, i2, r1, r2, sem1, sem2):
        wid = _wid()
        base = jnp.where(wid == NW - 1, N - ND_L, wid * ND_SP)
        pltpu.sync_copy(s1_hbm.at[pl.ds(base, ND_L)], i1)
        pltpu.sync_copy(s2_hbm.at[pl.ds(base, ND_L)], i2)
        cp1 = pltpu.async_copy(xl_hbm.at[i1], r1, sem1)
        cp2 = pltpu.async_copy(xl_hbm.at[i2], r2, sem2)
        cp1.wait()
        cp2.wait()
        pltpu.sync_copy(r1, g1_o.at[pl.ds(base, ND_L)])
        pltpu.sync_copy(r2, g2_o.at[pl.ds(base, ND_L)])

    out_type = [
        jax.ShapeDtypeStruct((N, 128), jnp.float32),
        jax.ShapeDtypeStruct((N, 128), jnp.float32),
    ]
    scratch = [
        pltpu.VMEM((ND_L,), jnp.int32), pltpu.VMEM((ND_L,), jnp.int32),
        pltpu.VMEM((ND_L, 128), jnp.float32),
        pltpu.VMEM((ND_L, 128), jnp.float32),
        pltpu.SemaphoreType.DMA, pltpu.SemaphoreType.DMA,
    ]
    fn = functools.partial(pl.kernel, mesh=_mesh(), out_type=out_type,
                           scratch_types=scratch)(kern)
    return fn(xl, s1, s2)


def _tc_combine(p, g1, g2, w1c, w2c, cntc, w_b, beta, plane, act):
    h = w_b.shape[1]
    hcat = p.shape[2]
    blk = 2000

    def kern(p_r, g1_r, g2_r, w1_r, w2_r, cnt_r, wb_r, beta_r, o_ref):
        out0 = p_r[plane, :, 0:h] + wb_r[...]
        num = (w1_r[...] * g1_r[:, 0:h] + w2_r[...] * g2_r[:, 0:h])
        out1 = num / jnp.maximum(cnt_r[...], 1.0)
        b = beta_r[0, 0]
        hh = b * out0 + (1.0 - b) * out1
        if act == "relu":
            o_ref[...] = jnp.maximum(hh, 0.0)
        else:
            m = jnp.max(hh, axis=1, keepdims=True)
            ex = jnp.exp(hh - m)
            o_ref[...] = (hh - m) - jnp.log(jnp.sum(ex, axis=1, keepdims=True))

    return pl.pallas_call(
        kern,
        grid=(N // blk,),
        in_specs=[
            pl.BlockSpec((2, blk, hcat), lambda i: (0, i, 0)),
            pl.BlockSpec((blk, 128), lambda i: (i, 0)),
            pl.BlockSpec((blk, 128), lambda i: (i, 0)),
            pl.BlockSpec((blk, 1), lambda i: (i, 0)),
            pl.BlockSpec((blk, 1), lambda i: (i, 0)),
            pl.BlockSpec((blk, 1), lambda i: (i, 0)),
            pl.BlockSpec((1, h), lambda i: (0, 0)),
            pl.BlockSpec((1, 1), lambda i: (0, 0)),
        ],
        out_specs=pl.BlockSpec((blk, h), lambda i: (i, 0)),
        out_shape=jax.ShapeDtypeStruct((N, h), jnp.float32),
    )(p, g1, g2, w1c, w2c, cntc, w_b, beta)


def kernel(x, edge_index, lin_w0, lin_b0, w_w0, w_b0, beta0,
           lin_w1, lin_b1, w_w1, w_b1, beta1):
    src = edge_index[0]
    dst = edge_index[1]
    h0 = lin_w0.shape[0]
    h1 = lin_w1.shape[0]

    dg = dst.reshape(E // 16, 16)
    dd = jnp.abs(dg[:, :, None] - dg[:, None, :]) <= 1
    eye = jnp.eye(16, dtype=bool)
    conf = jnp.any(dd & ~eye[None], axis=(1, 2)).astype(jnp.int32)
    flg = jnp.pad(conf.reshape(NW, EPW // 16), ((0, 0), (0, 640 - EPW // 16)))

    xl0, nrm0 = _tc_linnorm(x, lin_w0, lin_b0)
    tab0, smin = _sc_sim_select(nrm0, src, dst, flg, h0)
    b1v, b1e, b1s, b2v, b2e, b2s, cnt = _unpack_tab(tab0)
    w1, s1, w2, s2, cnt_m = _tc_merge(b1v, b1e, b1s, b2v, b2e, b2s, cnt)
    cntc = cnt_m.reshape(N, 1)

    row2d = _tc_rowidx(src.reshape(E // 128, 128), dst.reshape(E // 128, 128),
                       smin)
    w1tp = jnp.pad(w_w1.T, ((0, 0), (0, 128 - h1)))
    p = _sc_out0(w_w0.T, w1tp, dst, row2d.reshape(E))

    g1, g2 = _sc_gather_rows(xl0, s1.reshape(N), s2.reshape(N))
    hmid = _tc_combine(p, g1, g2, w1.reshape(N, 1), w2.reshape(N, 1), cntc,
                       w_b0.reshape(1, h0), beta0.reshape(1, 1), 0, "relu")

    xl1, nrm1 = _tc_linnorm(hmid, lin_w1, lin_b1)
    tab1, _ = _sc_sim_select(nrm1, src, dst, flg, h1)
    c1v, c1e, c1s, c2v, c2e, c2s, _ = _unpack_tab(tab1)
    v1, t1, v2, t2, _ = _tc_merge(c1v, c1e, c1s, c2v, c2e, c2s, cnt)
    k1, k2 = _sc_gather_rows(xl1, t1.reshape(N), t2.reshape(N))
    out = _tc_combine(p, k1, k2, v1.reshape(N, 1), v2.reshape(N, 1), cntc,
                      w_b1.reshape(1, h1), beta1.reshape(1, 1),
                      1, "softmax")
    return out
